# Initial kernel scaffold; baseline (speedup 1.0000x reference)
import jax
import jax.numpy as jnp
from jax.experimental import pallas as pl


def kernel(x, edge_index, batch, Wp, bp, gat0_W, gat0_as, gat0_ad, gat0_b, bn0_g, bn0_b, bn0_m, bn0_v, gat1_W, gat1_as, gat1_ad, gat1_b, bn1_g, bn1_b, bn1_m, bn1_v, gat2_W, gat2_as, gat2_ad, gat2_b, bn2_g, bn2_b, bn2_m, bn2_v, W1, b1, W2, b2, W3, b3):
    raise NotImplementedError("write your pallas kernel here")


# speedup vs baseline: 14.2974x; 14.2974x over previous
import functools
import jax
import jax.numpy as jnp
from jax.experimental import pallas as pl
from jax.experimental.pallas import tpu as pltpu

N = 50000
E = 800000
G = 16
IN = 9
H = 64
HEADS = 8
D = 8

_BN = 10000
_INTERPRET = False


def _row_spec(cols):
    return pl.BlockSpec((_BN, cols), lambda i: (i, 0))


def _full_spec(shape):
    return pl.BlockSpec(shape, lambda i: (0, 0))


def _proj_body(x_ref, wp_ref, bp_ref, h_ref):
    h_ref[...] = jnp.maximum(x_ref[...] @ wp_ref[...] + bp_ref[...], 0.0)


def _proj(x, wp, bp):
    return pl.pallas_call(
        _proj_body,
        grid=(N // _BN,),
        in_specs=[_row_spec(IN), _full_spec((IN, H)), _full_spec((1, H))],
        out_specs=_row_spec(H),
        out_shape=jax.ShapeDtypeStruct((N, H), jnp.float32),
        interpret=_INTERPRET,
    )(x, wp, bp.reshape(1, H))


def _k1_body(h_ref, w_ref, as_ref, ad_ref, hp_ref, als_ref, ald_ref,
             bmax_ref):
    hp = h_ref[...] @ w_ref[...]
    hp_ref[...] = hp
    als = hp @ as_ref[...]
    als_ref[...] = als
    ald_ref[...] = hp @ ad_ref[...]
    bmax_ref[...] = jnp.max(als, axis=0, keepdims=True)[None]


def _k1(h, w, a_s_flat, a_d_flat):
    return pl.pallas_call(
        _k1_body,
        grid=(N // _BN,),
        in_specs=[_row_spec(H), _full_spec((H, H)), _full_spec((H, HEADS)),
                  _full_spec((H, HEADS))],
        out_specs=[_row_spec(H), _row_spec(HEADS), _row_spec(HEADS),
                   pl.BlockSpec((1, 1, HEADS), lambda i: (i, 0, 0))],
        out_shape=[
            jax.ShapeDtypeStruct((N, H), jnp.float32),
            jax.ShapeDtypeStruct((N, HEADS), jnp.float32),
            jax.ShapeDtypeStruct((N, HEADS), jnp.float32),
            jax.ShapeDtypeStruct((N // _BN, 1, HEADS), jnp.float32),
        ],
        interpret=_INTERPRET,
    )(h, w, a_s_flat, a_d_flat)


def _mhat_body(bmax_ref, ald_ref, mhat_ref):
    z = jnp.max(bmax_ref[...], axis=(0, 1), keepdims=False)[None] + ald_ref[...]
    mhat_ref[...] = jnp.maximum(z, 0.2 * z)


def _mhat(bmax, ald):
    return pl.pallas_call(
        _mhat_body,
        grid=(N // _BN,),
        in_specs=[pl.BlockSpec((N // _BN, 1, HEADS), lambda i: (0, 0, 0)),
                  _row_spec(HEADS)],
        out_specs=_row_spec(HEADS),
        out_shape=jax.ShapeDtypeStruct((N, HEADS), jnp.float32),
        interpret=_INTERPRET,
    )(bmax, ald)


def _node_body(acc_ref, srep_ref, r_ref, b_ref, scale_ref, shift_ref, h_ref):
    out = acc_ref[...] / (srep_ref[...] + 1e-16) + b_ref[...]
    h_ref[...] = out * scale_ref[...] + shift_ref[...] + r_ref[...]


def _node(acc, srep, r, b, scale, shift):
    return pl.pallas_call(
        _node_body,
        grid=(N // _BN,),
        in_specs=[_row_spec(H), _row_spec(H), _row_spec(H),
                  _full_spec((1, H)), _full_spec((1, H)), _full_spec((1, H))],
        out_specs=_row_spec(H),
        out_shape=jax.ShapeDtypeStruct((N, H), jnp.float32),
        interpret=_INTERPRET,
    )(acc, srep, r, b.reshape(1, H), scale, shift)


def _pool_body(h_ref, batch_ref, w1_ref, b1_ref, w2_ref, b2_ref, w3_ref,
               b3_ref, out_ref, acc_ref, cnt_ref):
    i = pl.program_id(0)

    @pl.when(i == 0)
    def _():
        acc_ref[...] = jnp.zeros_like(acc_ref)
        cnt_ref[...] = jnp.zeros_like(cnt_ref)

    onehot = (batch_ref[...] == jax.lax.broadcasted_iota(
        jnp.int32, (1, G), 1)).astype(jnp.float32)
    dn = (((0,), (0,)), ((), ()))
    acc_ref[...] += jax.lax.dot_general(onehot, h_ref[...], dn)
    cnt_ref[...] += jax.lax.dot_general(
        onehot, jnp.zeros((_BN, 1), jnp.float32) + 1.0, dn)

    @pl.when(i == N // _BN - 1)
    def _():
        pooled = acc_ref[...] / jnp.maximum(cnt_ref[...], 1.0)
        z = jnp.maximum(pooled @ w1_ref[...] + b1_ref[...], 0.0)
        z = jnp.maximum(z @ w2_ref[...] + b2_ref[...], 0.0)
        out_ref[...] = z @ w3_ref[...] + b3_ref[...]


def _pool(h, batch, w1, b1, w2, b2, w3, b3):
    return pl.pallas_call(
        _pool_body,
        grid=(N // _BN,),
        in_specs=[_row_spec(H), _row_spec(1), _full_spec((H, H // 2)),
                  _full_spec((1, H // 2)), _full_spec((H // 2, H // 4)),
                  _full_spec((1, H // 4)), _full_spec((H // 4, 2)),
                  _full_spec((1, 2))],
        out_specs=_full_spec((G, 2)),
        out_shape=jax.ShapeDtypeStruct((G, 2), jnp.float32),
        scratch_shapes=[pltpu.VMEM((G, H), jnp.float32),
                        pltpu.VMEM((G, 1), jnp.float32)],
        interpret=_INTERPRET,
    )(h, batch.reshape(N, 1), w1, b1.reshape(1, H // 2), w2,
      b2.reshape(1, H // 4), w3, b3.reshape(1, 2))


def _expand_attn(a):
    return (a[:, :, None] * jnp.eye(HEADS, dtype=a.dtype)[:, None, :]).reshape(
        H, HEADS)


def _edge_phase(hp, als, ald, mhat, src, dst):
    z = als[src] + ald[dst]
    e = jnp.maximum(z, 0.2 * z)
    ex = jnp.exp(e - mhat[dst])
    s = jax.ops.segment_sum(ex, dst, num_segments=N)
    contrib = hp[src] * jnp.repeat(ex, D, axis=1)
    acc = jax.ops.segment_sum(contrib, dst, num_segments=N)
    return acc, jnp.repeat(s, D, axis=1)


def kernel(x, edge_index, batch, Wp, bp,
           gat0_W, gat0_as, gat0_ad, gat0_b, bn0_g, bn0_b, bn0_m, bn0_v,
           gat1_W, gat1_as, gat1_ad, gat1_b, bn1_g, bn1_b, bn1_m, bn1_v,
           gat2_W, gat2_as, gat2_ad, gat2_b, bn2_g, bn2_b, bn2_m, bn2_v,
           W1, b1, W2, b2, W3, b3):
    src, dst = edge_index[0], edge_index[1]
    h = _proj(x, Wp, bp)
    layers = [
        (gat0_W, gat0_as, gat0_ad, gat0_b, bn0_g, bn0_b, bn0_m, bn0_v),
        (gat1_W, gat1_as, gat1_ad, gat1_b, bn1_g, bn1_b, bn1_m, bn1_v),
        (gat2_W, gat2_as, gat2_ad, gat2_b, bn2_g, bn2_b, bn2_m, bn2_v),
    ]
    for (w, a_s, a_d, b, g, be, mu, var) in layers:
        scale = (g / jnp.sqrt(var + 1e-5)).reshape(1, H)
        shift = (be - mu * (g / jnp.sqrt(var + 1e-5))).reshape(1, H)
        hp, als, ald, bmax = _k1(h, w, _expand_attn(a_s), _expand_attn(a_d))
        mhat = _mhat(bmax, ald)
        acc, srep = _edge_phase(hp, als, ald, mhat, src, dst)
        h = _node(acc, srep, h, b, scale, shift)
    return _pool(h, batch, W1, b1, W2, b2, W3, b3)
